# baseline (speedup 1.0000x reference)
import functools

import numpy as np

import jax
import jax.numpy as jnp
from jax import lax
from jax.experimental import pallas as pl
from jax.experimental.pallas import tpu as pltpu
from jax.experimental.pallas import tpu_sc as plsc

N, D, H, E, K = 2048, 768, 3072, 8, 2
T = 256
NT = (N * K) // T + E
P = NT * T
HB = 3072
NHB = H // HB
NC, NS = 2, 16
NW = NC * NS
CH = (P // NW) // 3
CW = N // NW


_CHUNK = 128
_NCHUNK = N // _CHUNK

with jax.default_device(jax.local_devices(backend="cpu")[0]):
    _NOISE = np.asarray(
        jax.random.normal(jax.random.key(42), (N, E), dtype=jnp.float32)) * 0.1


def _gate_body(x_ref, wg_ref, bg_ref, noise_ref, p0_ref, p1_ref, w0_ref,
               w1_ref, te_ref, m_ref, s_ref):
    logits = lax.dot_general(
        x_ref[...], wg_ref[...], (((1,), (0,)), ((), ())),
        preferred_element_type=jnp.float32)
    logits = logits + bg_ref[...] + noise_ref[...]
    col = lax.broadcasted_iota(jnp.int32, (N, E), 1)
    m0 = jnp.max(logits, axis=1, keepdims=True)
    i0 = jnp.min(jnp.where(logits == m0, col, E), axis=1, keepdims=True)
    l2 = jnp.where(col == i0, -jnp.inf, logits)
    m1 = jnp.max(l2, axis=1, keepdims=True)
    i1 = jnp.min(jnp.where(l2 == m1, col, E), axis=1, keepdims=True)
    b = jnp.exp(m1 - m0)
    s = 1.0 + b
    w0_ref[...] = jnp.broadcast_to(1.0 / s, (N, 16))
    w1_ref[...] = jnp.broadcast_to(b / s, (N, 16))

    ohA = (col == i0).astype(jnp.float32)
    ohB = (col == i1).astype(jnp.float32)
    m_ref[...] = ohA + ohB
    ri = lax.broadcasted_iota(jnp.int32, (_CHUNK, _CHUNK), 0)
    rj = lax.broadcasted_iota(jnp.int32, (_CHUNK, _CHUNK), 1)
    tri = (rj < ri).astype(jnp.float32)

    def chunk_body(c, off):
        sl = pl.ds(c * _CHUNK, _CHUNK)
        chunk = m_ref[sl, :]
        within = lax.dot_general(tri, chunk, (((1,), (0,)), ((), ())),
                                 precision=lax.Precision.HIGHEST,
                                 preferred_element_type=jnp.float32)
        s_ref[sl, :] = within + off
        return off + jnp.sum(chunk, axis=0, keepdims=True)

    counts = lax.fori_loop(0, _NCHUNK, chunk_body,
                           jnp.zeros((1, E), jnp.float32))
    capt = jnp.floor((counts + (T - 1)) * (1.0 / T))
    ei = lax.broadcasted_iota(jnp.int32, (E, E), 0)
    ej = lax.broadcasted_iota(jnp.int32, (E, E), 1)
    trie = (ei < ej).astype(jnp.float32)
    ts = lax.dot_general(capt, trie, (((1,), (0,)), ((), ())),
                         precision=lax.Precision.HIGHEST,
                         preferred_element_type=jnp.float32)
    start = ts * T
    S = s_ref[...]
    pos0 = jnp.sum(ohA * (start + S), axis=1, keepdims=True)
    pos1 = jnp.sum(ohB * (start + S), axis=1, keepdims=True)
    p0_ref[...] = pos0.astype(jnp.int32)
    p1_ref[...] = pos1.astype(jnp.int32)

    bound = ts + capt
    tt = lax.broadcasted_iota(jnp.int32, (_CHUNK, E), 0).astype(jnp.float32)
    ge = (tt >= bound).astype(jnp.float32)
    te = jnp.minimum(jnp.sum(ge, axis=1, keepdims=True), E - 1.0)
    te_ref[...] = te.astype(jnp.int32)


def _gate(x, Wg, bg2, noise):
    return pl.pallas_call(
        _gate_body,
        out_shape=(
            jax.ShapeDtypeStruct((N, 1), jnp.int32),
            jax.ShapeDtypeStruct((N, 1), jnp.int32),
            jax.ShapeDtypeStruct((N, 16), jnp.float32),
            jax.ShapeDtypeStruct((N, 16), jnp.float32),
            jax.ShapeDtypeStruct((_CHUNK, 1), jnp.int32),
        ),
        scratch_shapes=[
            pltpu.VMEM((N, E), jnp.float32),
            pltpu.VMEM((N, E), jnp.float32),
        ],
    )(x, Wg, bg2, noise)


@functools.partial(
    pl.kernel,
    mesh=plsc.VectorSubcoreMesh(core_axis_name="c", subcore_axis_name="s"),
    out_type=jax.ShapeDtypeStruct((P, D), jnp.float32),
    scratch_types=[
        pltpu.VMEM((CW,), jnp.int32),
        pltpu.VMEM((CW,), jnp.int32),
        pltpu.VMEM((CW, D), jnp.float32),
        pltpu.SemaphoreType.DMA,
        pltpu.SemaphoreType.DMA,
        pltpu.SemaphoreType.DMA,
    ],
)
def _dispatch(x_hbm, p0_hbm, p1_hbm, out_hbm, i0_v, i1_v, xb_v,
              s0, s1, s2):
    wid = lax.axis_index("s") * NC + lax.axis_index("c")
    base = wid * CW
    l0 = pltpu.async_copy(p0_hbm.at[pl.ds(base, CW)], i0_v, s0)
    l1 = pltpu.async_copy(p1_hbm.at[pl.ds(base, CW)], i1_v, s1)
    l2 = pltpu.async_copy(x_hbm.at[pl.ds(base, CW)], xb_v, s2)
    l0.wait()
    l1.wait()
    l2.wait()
    c0 = pltpu.async_copy(xb_v, out_hbm.at[i0_v], s0)
    c1 = pltpu.async_copy(xb_v, out_hbm.at[i1_v], s1)
    c0.wait()
    c1.wait()


def _mlp_body(te_ref, xg_ref, w1_ref, w2_ref, b1_ref, b2_ref, out_ref):
    xb = xg_ref[...].astype(jnp.bfloat16)
    w1 = w1_ref[0].astype(jnp.bfloat16)
    hp = lax.dot_general(xb, w1, (((1,), (0,)), ((), ())),
                         preferred_element_type=jnp.float32)
    hp = jnp.maximum(hp + b1_ref[0], 0.0).astype(jnp.bfloat16)
    w2 = w2_ref[0].astype(jnp.bfloat16)
    contrib = lax.dot_general(hp, w2, (((1,), (0,)), ((), ())),
                              preferred_element_type=jnp.float32)
    out_ref[...] = contrib + b2_ref[0]


def _mlp(tile_e, xg, W1, W2, b1r, b2r):
    grid_spec = pltpu.PrefetchScalarGridSpec(
        num_scalar_prefetch=1,
        grid=(NT,),
        in_specs=[
            pl.BlockSpec((T, D), lambda t, te: (t, 0)),
            pl.BlockSpec((1, D, H), lambda t, te: (te[t], 0, 0)),
            pl.BlockSpec((1, H, D), lambda t, te: (te[t], 0, 0)),
            pl.BlockSpec((1, 1, H), lambda t, te: (te[t], 0, 0)),
            pl.BlockSpec((1, 1, D), lambda t, te: (te[t], 0, 0)),
        ],
        out_specs=pl.BlockSpec((T, D), lambda t, te: (t, 0)),
    )
    return pl.pallas_call(
        _mlp_body,
        grid_spec=grid_spec,
        out_shape=jax.ShapeDtypeStruct((P, D), jnp.float32),
    )(tile_e, xg, W1, W2, b1r, b2r)


@functools.partial(
    pl.kernel,
    mesh=plsc.VectorSubcoreMesh(core_axis_name="c", subcore_axis_name="s"),
    out_type=jax.ShapeDtypeStruct((N, D), jnp.float32),
    scratch_types=[
        pltpu.VMEM((CW,), jnp.int32),
        pltpu.VMEM((CW,), jnp.int32),
        pltpu.VMEM((CW, 16), jnp.float32),
        pltpu.VMEM((CW, 16), jnp.float32),
        pltpu.VMEM((CW // 2, D), jnp.float32),
        pltpu.VMEM((CW // 2, D), jnp.float32),
        pltpu.VMEM((CW // 2, D), jnp.float32),
        pltpu.VMEM((CW // 2, D), jnp.float32),
        pltpu.SemaphoreType.DMA,
        pltpu.SemaphoreType.DMA,
        pltpu.SemaphoreType.DMA,
        pltpu.SemaphoreType.DMA,
    ],
)
def _combine(y_hbm, pa_hbm, pb_hbm, wa_hbm, wb_hbm, out_hbm,
             ia_v, ib_v, wa_v, wb_v, ra0_v, rb0_v, ra1_v, rb1_v,
             sa0, sb0, sa1, sb1):
    wid = lax.axis_index("s") * NC + lax.axis_index("c")
    base = wid * CW
    hw = CW // 2
    pltpu.sync_copy(pa_hbm.at[pl.ds(base, CW)], ia_v)
    pltpu.sync_copy(pb_hbm.at[pl.ds(base, CW)], ib_v)
    pltpu.sync_copy(wa_hbm.at[pl.ds(base, CW)], wa_v)
    pltpu.sync_copy(wb_hbm.at[pl.ds(base, CW)], wb_v)
    g0 = pltpu.async_copy(y_hbm.at[ia_v.at[pl.ds(0, hw)]], ra0_v, sa0)
    g1 = pltpu.async_copy(y_hbm.at[ib_v.at[pl.ds(0, hw)]], rb0_v, sb0)
    g2 = pltpu.async_copy(y_hbm.at[ia_v.at[pl.ds(hw, hw)]], ra1_v, sa1)
    g3 = pltpu.async_copy(y_hbm.at[ib_v.at[pl.ds(hw, hw)]], rb1_v, sb1)

    def half(ra, rb, woff):
        def row_body(j, carry):
            wa = wa_v[woff + j, :]
            wb = wb_v[woff + j, :]
            for k in range(D // 16):
                sl = pl.ds(k * 16, 16)
                ra[j, sl] = wa * ra[j, sl] + wb * rb[j, sl]
            return carry
        lax.fori_loop(0, hw, row_body, 0)

    g0.wait()
    g1.wait()
    half(ra0_v, rb0_v, 0)
    st0 = pltpu.async_copy(ra0_v, out_hbm.at[pl.ds(base, hw)], sa0)
    g2.wait()
    g3.wait()
    half(ra1_v, rb1_v, hw)
    st0.wait()
    pltpu.sync_copy(ra1_v, out_hbm.at[pl.ds(base + hw, hw)])


def kernel(x, Wg, bg, W1, b1, W2, b2):
    p0, p1, w0, w1, te128 = _gate(x, Wg, bg.reshape(1, E),
                                  jnp.asarray(_NOISE))
    tile_e = te128[:NT, 0]

    xg = _dispatch(x, p0[:, 0], p1[:, 0])
    y = _mlp(tile_e, xg, W1, W2, b1.reshape(E, 1, H), b2.reshape(E, 1, D))
    out = _combine(y, p0[:, 0], p1[:, 0], w0, w1)
    return out


# speedup vs baseline: 1.0460x; 1.0460x over previous
import functools

import numpy as np

import jax
import jax.numpy as jnp
from jax import lax
from jax.experimental import pallas as pl
from jax.experimental.pallas import tpu as pltpu
from jax.experimental.pallas import tpu_sc as plsc

N, D, H, E, K = 2048, 768, 3072, 8, 2
T = 256
NT = (N * K) // T + E
P = NT * T
HB = 3072
NHB = H // HB
NC, NS = 2, 16
NW = NC * NS
CH = (P // NW) // 3
CW = N // NW


_CHUNK = 128
_NCHUNK = N // _CHUNK

with jax.default_device(jax.local_devices(backend="cpu")[0]):
    _NOISE = np.asarray(
        jax.random.normal(jax.random.key(42), (N, E), dtype=jnp.float32)) * 0.1


def _gate_body(x_ref, wg_ref, bg_ref, noise_ref, p0_ref, p1_ref, w0_ref,
               w1_ref, te_ref, m_ref, s_ref):
    logits = lax.dot_general(
        x_ref[...], wg_ref[...], (((1,), (0,)), ((), ())),
        preferred_element_type=jnp.float32)
    logits = logits + bg_ref[...] + noise_ref[...]
    col = lax.broadcasted_iota(jnp.int32, (N, E), 1)
    m0 = jnp.max(logits, axis=1, keepdims=True)
    i0 = jnp.min(jnp.where(logits == m0, col, E), axis=1, keepdims=True)
    l2 = jnp.where(col == i0, -jnp.inf, logits)
    m1 = jnp.max(l2, axis=1, keepdims=True)
    i1 = jnp.min(jnp.where(l2 == m1, col, E), axis=1, keepdims=True)
    b = jnp.exp(m1 - m0)
    s = 1.0 + b
    w0_ref[...] = jnp.broadcast_to(1.0 / s, (N, 16))
    w1_ref[...] = jnp.broadcast_to(b / s, (N, 16))

    ohA = (col == i0).astype(jnp.float32)
    ohB = (col == i1).astype(jnp.float32)
    m_ref[...] = ohA + ohB
    ri = lax.broadcasted_iota(jnp.int32, (_CHUNK, _CHUNK), 0)
    rj = lax.broadcasted_iota(jnp.int32, (_CHUNK, _CHUNK), 1)
    tri = (rj < ri).astype(jnp.float32)

    def chunk_body(c, off):
        sl = pl.ds(c * _CHUNK, _CHUNK)
        chunk = m_ref[sl, :]
        within = lax.dot_general(tri, chunk, (((1,), (0,)), ((), ())),
                                 precision=lax.Precision.HIGHEST,
                                 preferred_element_type=jnp.float32)
        s_ref[sl, :] = within + off
        return off + jnp.sum(chunk, axis=0, keepdims=True)

    counts = lax.fori_loop(0, _NCHUNK, chunk_body,
                           jnp.zeros((1, E), jnp.float32))
    capt = jnp.floor((counts + (T - 1)) * (1.0 / T))
    ei = lax.broadcasted_iota(jnp.int32, (E, E), 0)
    ej = lax.broadcasted_iota(jnp.int32, (E, E), 1)
    trie = (ei < ej).astype(jnp.float32)
    ts = lax.dot_general(capt, trie, (((1,), (0,)), ((), ())),
                         precision=lax.Precision.HIGHEST,
                         preferred_element_type=jnp.float32)
    start = ts * T
    S = s_ref[...]
    pos0 = jnp.sum(ohA * (start + S), axis=1, keepdims=True)
    pos1 = jnp.sum(ohB * (start + S), axis=1, keepdims=True)
    p0_ref[...] = pos0.astype(jnp.int32)
    p1_ref[...] = pos1.astype(jnp.int32)

    bound = ts + capt
    tt = lax.broadcasted_iota(jnp.int32, (_CHUNK, E), 0).astype(jnp.float32)
    ge = (tt >= bound).astype(jnp.float32)
    te = jnp.minimum(jnp.sum(ge, axis=1, keepdims=True), E - 1.0)
    ntr = jnp.sum(capt)
    rowi = lax.broadcasted_iota(jnp.int32, (_CHUNK, 1), 0)
    te = jnp.where(rowi == NT, ntr, te)
    te_ref[...] = te.astype(jnp.int32)


def _gate(x, Wg, bg2, noise):
    return pl.pallas_call(
        _gate_body,
        out_shape=(
            jax.ShapeDtypeStruct((N, 1), jnp.int32),
            jax.ShapeDtypeStruct((N, 1), jnp.int32),
            jax.ShapeDtypeStruct((N, 16), jnp.float32),
            jax.ShapeDtypeStruct((N, 16), jnp.float32),
            jax.ShapeDtypeStruct((_CHUNK, 1), jnp.int32),
        ),
        scratch_shapes=[
            pltpu.VMEM((N, E), jnp.float32),
            pltpu.VMEM((N, E), jnp.float32),
        ],
    )(x, Wg, bg2, noise)


@functools.partial(
    pl.kernel,
    mesh=plsc.VectorSubcoreMesh(core_axis_name="c", subcore_axis_name="s"),
    out_type=jax.ShapeDtypeStruct((P, D), jnp.float32),
    scratch_types=[
        pltpu.VMEM((CW,), jnp.int32),
        pltpu.VMEM((CW,), jnp.int32),
        pltpu.VMEM((CW, D), jnp.float32),
        pltpu.SemaphoreType.DMA,
        pltpu.SemaphoreType.DMA,
        pltpu.SemaphoreType.DMA,
    ],
)
def _dispatch(x_hbm, p0_hbm, p1_hbm, out_hbm, i0_v, i1_v, xb_v,
              s0, s1, s2):
    wid = lax.axis_index("s") * NC + lax.axis_index("c")
    base = wid * CW
    l0 = pltpu.async_copy(p0_hbm.at[pl.ds(base, CW)], i0_v, s0)
    l1 = pltpu.async_copy(p1_hbm.at[pl.ds(base, CW)], i1_v, s1)
    l2 = pltpu.async_copy(x_hbm.at[pl.ds(base, CW)], xb_v, s2)
    l0.wait()
    l1.wait()
    l2.wait()
    c0 = pltpu.async_copy(xb_v, out_hbm.at[i0_v], s0)
    c1 = pltpu.async_copy(xb_v, out_hbm.at[i1_v], s1)
    c0.wait()
    c1.wait()


def _mlp_body(te_ref, xg_ref, w1_ref, w2_ref, b1_ref, b2_ref, out_ref):
    t = pl.program_id(0)

    @pl.when(t < te_ref[NT])
    def _():
        xb = xg_ref[...].astype(jnp.bfloat16)
        w1 = w1_ref[0].astype(jnp.bfloat16)
        hp = lax.dot_general(xb, w1, (((1,), (0,)), ((), ())),
                             preferred_element_type=jnp.float32)
        hp = jnp.maximum(hp + b1_ref[0], 0.0).astype(jnp.bfloat16)
        w2 = w2_ref[0].astype(jnp.bfloat16)
        contrib = lax.dot_general(hp, w2, (((1,), (0,)), ((), ())),
                                  preferred_element_type=jnp.float32)
        out_ref[...] = contrib + b2_ref[0]


def _mlp(tile_e, xg, W1, W2, b1r, b2r):
    grid_spec = pltpu.PrefetchScalarGridSpec(
        num_scalar_prefetch=1,
        grid=(NT,),
        in_specs=[
            pl.BlockSpec((T, D), lambda t, te: (t, 0)),
            pl.BlockSpec((1, D, H), lambda t, te: (te[t], 0, 0)),
            pl.BlockSpec((1, H, D), lambda t, te: (te[t], 0, 0)),
            pl.BlockSpec((1, 1, H), lambda t, te: (te[t], 0, 0)),
            pl.BlockSpec((1, 1, D), lambda t, te: (te[t], 0, 0)),
        ],
        out_specs=pl.BlockSpec((T, D), lambda t, te: (t, 0)),
    )
    return pl.pallas_call(
        _mlp_body,
        grid_spec=grid_spec,
        out_shape=jax.ShapeDtypeStruct((P, D), jnp.float32),
    )(tile_e, xg, W1, W2, b1r, b2r)


@functools.partial(
    pl.kernel,
    mesh=plsc.VectorSubcoreMesh(core_axis_name="c", subcore_axis_name="s"),
    out_type=jax.ShapeDtypeStruct((N, D), jnp.float32),
    scratch_types=[
        pltpu.VMEM((CW,), jnp.int32),
        pltpu.VMEM((CW,), jnp.int32),
        pltpu.VMEM((CW, 16), jnp.float32),
        pltpu.VMEM((CW, 16), jnp.float32),
        pltpu.VMEM((CW, D), jnp.float32),
        pltpu.VMEM((CW, D), jnp.float32),
        pltpu.SemaphoreType.DMA,
        pltpu.SemaphoreType.DMA,
    ],
)
def _combine(y_hbm, pa_hbm, pb_hbm, wa_hbm, wb_hbm, out_hbm,
             ia_v, ib_v, wa_v, wb_v, ra_v, rb_v, sa, sb):
    wid = lax.axis_index("s") * NC + lax.axis_index("c")
    base = wid * CW
    pltpu.sync_copy(pa_hbm.at[pl.ds(base, CW)], ia_v)
    pltpu.sync_copy(pb_hbm.at[pl.ds(base, CW)], ib_v)
    pltpu.sync_copy(wa_hbm.at[pl.ds(base, CW)], wa_v)
    pltpu.sync_copy(wb_hbm.at[pl.ds(base, CW)], wb_v)
    cpa = pltpu.async_copy(y_hbm.at[ia_v], ra_v, sa)
    cpb = pltpu.async_copy(y_hbm.at[ib_v], rb_v, sb)
    cpa.wait()
    cpb.wait()

    def row_body(j, carry):
        wa = wa_v[j, :]
        wb = wb_v[j, :]
        for k in range(D // 16):
            sl = pl.ds(k * 16, 16)
            ra_v[j, sl] = wa * ra_v[j, sl] + wb * rb_v[j, sl]
        return carry

    lax.fori_loop(0, CW, row_body, 0)
    pltpu.sync_copy(ra_v, out_hbm.at[pl.ds(base, CW)])


def kernel(x, Wg, bg, W1, b1, W2, b2):
    p0, p1, w0, w1, te128 = _gate(x, Wg, bg.reshape(1, E),
                                  jnp.asarray(_NOISE))
    tile_e = te128[:NT + 1, 0]

    xg = _dispatch(x, p0[:, 0], p1[:, 0])
    y = _mlp(tile_e, xg, W1, W2, b1.reshape(E, 1, H), b2.reshape(E, 1, D))
    out = _combine(y, p0[:, 0], p1[:, 0], w0, w1)
    return out


# speedup vs baseline: 1.0498x; 1.0037x over previous
import functools

import numpy as np

import jax
import jax.numpy as jnp
from jax import lax
from jax.experimental import pallas as pl
from jax.experimental.pallas import tpu as pltpu
from jax.experimental.pallas import tpu_sc as plsc

N, D, H, E, K = 2048, 768, 3072, 8, 2
T = 256
NT = (N * K) // T + E
P = NT * T
HB = 3072
NHB = H // HB
NC, NS = 2, 16
NW = NC * NS
CH = (P // NW) // 3
CW = N // NW


_CHUNK = 256
_NCHUNK = N // _CHUNK

with jax.default_device(jax.local_devices(backend="cpu")[0]):
    _NOISE = np.asarray(
        jax.random.normal(jax.random.key(42), (N, E), dtype=jnp.float32)) * 0.1


def _gate_body(x_ref, wg_ref, bg_ref, noise_ref, p0_ref, p1_ref, w0_ref,
               w1_ref, te_ref, m_ref, s_ref):
    logits = lax.dot_general(
        x_ref[...], wg_ref[...], (((1,), (0,)), ((), ())),
        preferred_element_type=jnp.float32)
    logits = logits + bg_ref[...] + noise_ref[...]
    col = lax.broadcasted_iota(jnp.int32, (N, E), 1)
    m0 = jnp.max(logits, axis=1, keepdims=True)
    i0 = jnp.min(jnp.where(logits == m0, col, E), axis=1, keepdims=True)
    l2 = jnp.where(col == i0, -jnp.inf, logits)
    m1 = jnp.max(l2, axis=1, keepdims=True)
    i1 = jnp.min(jnp.where(l2 == m1, col, E), axis=1, keepdims=True)
    b = jnp.exp(m1 - m0)
    s = 1.0 + b
    w0_ref[...] = jnp.broadcast_to(1.0 / s, (N, 16))
    w1_ref[...] = jnp.broadcast_to(b / s, (N, 16))

    ohA = (col == i0).astype(jnp.float32)
    ohB = (col == i1).astype(jnp.float32)
    m_ref[...] = ohA + ohB
    ri = lax.broadcasted_iota(jnp.int32, (_CHUNK, _CHUNK), 0)
    rj = lax.broadcasted_iota(jnp.int32, (_CHUNK, _CHUNK), 1)
    tri = (rj < ri).astype(jnp.float32)

    def chunk_body(c, off):
        sl = pl.ds(c * _CHUNK, _CHUNK)
        chunk = m_ref[sl, :]
        within = lax.dot_general(tri, chunk, (((1,), (0,)), ((), ())),
                                 precision=lax.Precision.HIGHEST,
                                 preferred_element_type=jnp.float32)
        s_ref[sl, :] = within + off
        return off + jnp.sum(chunk, axis=0, keepdims=True)

    counts = lax.fori_loop(0, _NCHUNK, chunk_body,
                           jnp.zeros((1, E), jnp.float32))
    capt = jnp.floor((counts + (T - 1)) * (1.0 / T))
    ei = lax.broadcasted_iota(jnp.int32, (E, E), 0)
    ej = lax.broadcasted_iota(jnp.int32, (E, E), 1)
    trie = (ei < ej).astype(jnp.float32)
    ts = lax.dot_general(capt, trie, (((1,), (0,)), ((), ())),
                         precision=lax.Precision.HIGHEST,
                         preferred_element_type=jnp.float32)
    start = ts * T
    S = s_ref[...]
    pos0 = jnp.sum(ohA * (start + S), axis=1, keepdims=True)
    pos1 = jnp.sum(ohB * (start + S), axis=1, keepdims=True)
    p0_ref[...] = pos0.astype(jnp.int32)
    p1_ref[...] = pos1.astype(jnp.int32)

    bound = ts + capt
    tt = lax.broadcasted_iota(jnp.int32, (_CHUNK, E), 0).astype(jnp.float32)
    ge = (tt >= bound).astype(jnp.float32)
    te = jnp.minimum(jnp.sum(ge, axis=1, keepdims=True), E - 1.0)
    ntr = jnp.sum(capt)
    rowi = lax.broadcasted_iota(jnp.int32, (_CHUNK, 1), 0)
    te = jnp.where(rowi == NT, ntr, te)
    te_ref[...] = te.astype(jnp.int32)


def _gate(x, Wg, bg2, noise):
    return pl.pallas_call(
        _gate_body,
        out_shape=(
            jax.ShapeDtypeStruct((N, 1), jnp.int32),
            jax.ShapeDtypeStruct((N, 1), jnp.int32),
            jax.ShapeDtypeStruct((N, 16), jnp.float32),
            jax.ShapeDtypeStruct((N, 16), jnp.float32),
            jax.ShapeDtypeStruct((_CHUNK, 1), jnp.int32),
        ),
        scratch_shapes=[
            pltpu.VMEM((N, E), jnp.float32),
            pltpu.VMEM((N, E), jnp.float32),
        ],
    )(x, Wg, bg2, noise)


@functools.partial(
    pl.kernel,
    mesh=plsc.VectorSubcoreMesh(core_axis_name="c", subcore_axis_name="s"),
    out_type=jax.ShapeDtypeStruct((P, D), jnp.float32),
    scratch_types=[
        pltpu.VMEM((CW,), jnp.int32),
        pltpu.VMEM((CW,), jnp.int32),
        pltpu.VMEM((CW, D), jnp.float32),
        pltpu.SemaphoreType.DMA,
        pltpu.SemaphoreType.DMA,
        pltpu.SemaphoreType.DMA,
    ],
)
def _dispatch(x_hbm, p0_hbm, p1_hbm, out_hbm, i0_v, i1_v, xb_v,
              s0, s1, s2):
    wid = lax.axis_index("s") * NC + lax.axis_index("c")
    base = wid * CW
    l0 = pltpu.async_copy(p0_hbm.at[pl.ds(base, CW)], i0_v, s0)
    l1 = pltpu.async_copy(p1_hbm.at[pl.ds(base, CW)], i1_v, s1)
    l2 = pltpu.async_copy(x_hbm.at[pl.ds(base, CW)], xb_v, s2)
    l0.wait()
    l1.wait()
    l2.wait()
    c0 = pltpu.async_copy(xb_v, out_hbm.at[i0_v], s0)
    c1 = pltpu.async_copy(xb_v, out_hbm.at[i1_v], s1)
    c0.wait()
    c1.wait()


def _mlp_body(te_ref, xg_ref, w1_ref, w2_ref, b1_ref, b2_ref, out_ref):
    t = pl.program_id(0)

    @pl.when(t < te_ref[NT])
    def _():
        xb = xg_ref[...].astype(jnp.bfloat16)
        w1 = w1_ref[0].astype(jnp.bfloat16)
        hp = lax.dot_general(xb, w1, (((1,), (0,)), ((), ())),
                             preferred_element_type=jnp.float32)
        hp = jnp.maximum(hp + b1_ref[0], 0.0).astype(jnp.bfloat16)
        w2 = w2_ref[0].astype(jnp.bfloat16)
        contrib = lax.dot_general(hp, w2, (((1,), (0,)), ((), ())),
                                  preferred_element_type=jnp.float32)
        out_ref[...] = contrib + b2_ref[0]


def _mlp(tile_e, xg, W1, W2, b1r, b2r):
    grid_spec = pltpu.PrefetchScalarGridSpec(
        num_scalar_prefetch=1,
        grid=(NT,),
        in_specs=[
            pl.BlockSpec((T, D), lambda t, te: (t, 0)),
            pl.BlockSpec((1, D, H), lambda t, te: (te[t], 0, 0)),
            pl.BlockSpec((1, H, D), lambda t, te: (te[t], 0, 0)),
            pl.BlockSpec((1, 1, H), lambda t, te: (te[t], 0, 0)),
            pl.BlockSpec((1, 1, D), lambda t, te: (te[t], 0, 0)),
        ],
        out_specs=pl.BlockSpec((T, D), lambda t, te: (t, 0)),
    )
    return pl.pallas_call(
        _mlp_body,
        grid_spec=grid_spec,
        out_shape=jax.ShapeDtypeStruct((P, D), jnp.float32),
    )(tile_e, xg, W1, W2, b1r, b2r)


@functools.partial(
    pl.kernel,
    mesh=plsc.VectorSubcoreMesh(core_axis_name="c", subcore_axis_name="s"),
    out_type=jax.ShapeDtypeStruct((N, D), jnp.float32),
    scratch_types=[
        pltpu.VMEM((CW,), jnp.int32),
        pltpu.VMEM((CW,), jnp.int32),
        pltpu.VMEM((CW, 16), jnp.float32),
        pltpu.VMEM((CW, 16), jnp.float32),
        pltpu.VMEM((CW, D), jnp.float32),
        pltpu.VMEM((CW, D), jnp.float32),
        pltpu.SemaphoreType.DMA,
        pltpu.SemaphoreType.DMA,
    ],
)
def _combine(y_hbm, pa_hbm, pb_hbm, wa_hbm, wb_hbm, out_hbm,
             ia_v, ib_v, wa_v, wb_v, ra_v, rb_v, sa, sb):
    wid = lax.axis_index("s") * NC + lax.axis_index("c")
    base = wid * CW
    pltpu.sync_copy(pa_hbm.at[pl.ds(base, CW)], ia_v)
    pltpu.sync_copy(pb_hbm.at[pl.ds(base, CW)], ib_v)
    pltpu.sync_copy(wa_hbm.at[pl.ds(base, CW)], wa_v)
    pltpu.sync_copy(wb_hbm.at[pl.ds(base, CW)], wb_v)
    cpa = pltpu.async_copy(y_hbm.at[ia_v], ra_v, sa)
    cpb = pltpu.async_copy(y_hbm.at[ib_v], rb_v, sb)
    cpa.wait()
    cpb.wait()

    def row_body(j, carry):
        wa = wa_v[j, :]
        wb = wb_v[j, :]
        for k in range(D // 16):
            sl = pl.ds(k * 16, 16)
            ra_v[j, sl] = wa * ra_v[j, sl] + wb * rb_v[j, sl]
        return carry

    lax.fori_loop(0, CW, row_body, 0)
    pltpu.sync_copy(ra_v, out_hbm.at[pl.ds(base, CW)])


def kernel(x, Wg, bg, W1, b1, W2, b2):
    p0, p1, w0, w1, te128 = _gate(x, Wg, bg.reshape(1, E),
                                  jnp.asarray(_NOISE))
    tile_e = te128[:NT + 1, 0]

    xg = _dispatch(x, p0[:, 0], p1[:, 0])
    y = _mlp(tile_e, xg, W1, W2, b1.reshape(E, 1, H), b2.reshape(E, 1, D))
    out = _combine(y, p0[:, 0], p1[:, 0], w0, w1)
    return out
